# Initial kernel scaffold; baseline (speedup 1.0000x reference)
import jax
import jax.numpy as jnp
from jax.experimental import pallas as pl


def kernel(x, edge_index, W, att_src, att_dst, bias):
    raise NotImplementedError("write your pallas kernel here")


# speedup vs baseline: 31.5305x; 31.5305x over previous
import functools

import jax
import jax.numpy as jnp
from jax import lax
from jax.experimental import pallas as pl
from jax.experimental.pallas import tpu as pltpu
from jax.experimental.pallas import tpu_sc as plsc

N = 10000
E = 320000
F = 128
H = 8
C = 32
D = H * C
HALF = D // 2
HPC = H // 2
NSUB = 16
K = 80
NCH = (E // NSUB) // K
PW = HPC * K
BLK = 1000
RSL = 2512
DSTRIDE = 10048
DTOT = HPC * DSTRIDE


def _tc_project_body(x_ref, w_ref, as_ref, ad_ref, h0_ref, h1_ref,
                     als_ref, ald_ref):
    h = jnp.dot(x_ref[...], w_ref[...], preferred_element_type=jnp.float32)
    h0_ref[...] = h[:, :HALF]
    h1_ref[...] = h[:, HALF:]
    als_ref[...] = jnp.dot(h, as_ref[...], preferred_element_type=jnp.float32)
    ald_ref[...] = jnp.dot(h, ad_ref[...], preferred_element_type=jnp.float32)


def _tc_project(x, W, A_src, A_dst):
    return pl.pallas_call(
        _tc_project_body,
        grid=(N // BLK,),
        in_specs=[
            pl.BlockSpec((BLK, F), lambda i: (i, 0)),
            pl.BlockSpec((F, D), lambda i: (0, 0)),
            pl.BlockSpec((D, H), lambda i: (0, 0)),
            pl.BlockSpec((D, H), lambda i: (0, 0)),
        ],
        out_specs=[
            pl.BlockSpec((BLK, HALF), lambda i: (i, 0)),
            pl.BlockSpec((BLK, HALF), lambda i: (i, 0)),
            pl.BlockSpec((BLK, H), lambda i: (i, 0)),
            pl.BlockSpec((BLK, H), lambda i: (i, 0)),
        ],
        out_shape=[
            jax.ShapeDtypeStruct((N, HALF), jnp.float32),
            jax.ShapeDtypeStruct((N, HALF), jnp.float32),
            jax.ShapeDtypeStruct((N, H), jnp.float32),
            jax.ShapeDtypeStruct((N, H), jnp.float32),
        ],
    )(x, W, A_src, A_dst)


def _sc_mesh():
    return plsc.VectorSubcoreMesh(core_axis_name="c", subcore_axis_name="s")


@functools.cache
def _get_sc_weights():
    @functools.partial(
        pl.kernel,
        out_type=[
            jax.ShapeDtypeStruct((2 * E * HPC,), jnp.float32),
            jax.ShapeDtypeStruct((2 * DTOT,), jnp.float32),
            jax.ShapeDtypeStruct((2 * NSUB * DTOT,), jnp.float32),
        ],
        mesh=_sc_mesh(),
        scratch_types=[
            pltpu.VMEM((HPC * N,), jnp.float32),
            pltpu.VMEM((HPC * N,), jnp.float32),
            pltpu.VMEM((1, K), jnp.int32),
            pltpu.VMEM((1, K), jnp.int32),
            pltpu.VMEM((PW,), jnp.float32),
            pltpu.VMEM((DTOT,), jnp.float32),
        ],
        compiler_params=pltpu.CompilerParams(needs_layout_passes=False),
    )
    def _sc_weights(alsT, aldT, srcE, dstE, zflat,
                    p_out, den_out, den_part,
                    as_v, ad_v, si_v, di_v, pch, den_v):
        c = lax.axis_index("c")
        s = lax.axis_index("s")

        pltpu.sync_copy(alsT.at[pl.ds(c * (HPC * N), HPC * N)], as_v)
        pltpu.sync_copy(aldT.at[pl.ds(c * (HPC * N), HPC * N)], ad_v)
        pltpu.sync_copy(zflat, den_v)

        def chunk(i, carry):
            row = s * NCH + i
            pltpu.sync_copy(srcE.at[row], si_v)
            pltpu.sync_copy(dstE.at[row], di_v)
            for g in range(K // 16):
                s16 = si_v[0, pl.ds(g * 16, 16)]
                d16 = di_v[0, pl.ds(g * 16, 16)]
                for hh in range(HPC):
                    off = jnp.full((16,), hh * N, jnp.int32)
                    a_s = plsc.load_gather(as_v, [off + s16])
                    a_d = plsc.load_gather(ad_v, [off + d16])
                    z = a_s + a_d
                    e = jnp.maximum(z, 0.2 * z)
                    p = jnp.exp(e)
                    doff = jnp.full((16,), hh * DSTRIDE, jnp.int32)
                    plsc.addupdate_scatter(den_v, [doff + d16], p)
                    pch[pl.ds(hh * K + g * 16, 16)] = p
            pltpu.sync_copy(pch, p_out.at[pl.ds(((c * NSUB + s) * NCH + i) * PW, PW)])
            return carry

        lax.fori_loop(0, NCH, chunk, 0)

        pltpu.sync_copy(den_v, den_part.at[pl.ds((c * NSUB + s) * DTOT, DTOT)])
        plsc.subcore_barrier()

        base = s * RSL
        pltpu.sync_copy(den_part.at[pl.ds(c * NSUB * DTOT + base, RSL)],
                        den_v.at[pl.ds(RSL, RSL)])

        def red(k, carry):
            pltpu.sync_copy(
                den_part.at[pl.ds((c * NSUB + k) * DTOT + base, RSL)],
                den_v.at[pl.ds(0, RSL)])
            for v in range(RSL // 16):
                sa = pl.ds(v * 16, 16)
                sb = pl.ds(RSL + v * 16, 16)
                den_v[sb] = den_v[sb] + den_v[sa]
            return carry

        lax.fori_loop(1, NSUB, red, 0)

        pltpu.sync_copy(den_v.at[pl.ds(RSL, RSL)],
                        den_out.at[pl.ds(c * DTOT + base, RSL)])

    return _sc_weights


@functools.cache
def _get_sc_scatter():
    @functools.partial(
        pl.kernel,
        out_type=jax.ShapeDtypeStruct((2, N, HALF), jnp.float32),
        mesh=_sc_mesh(),
        scratch_types=[
            pltpu.VMEM((1, K), jnp.int32),
            pltpu.VMEM((1, K), jnp.int32),
            pltpu.VMEM((K, HALF), jnp.float32),
            pltpu.VMEM((PW,), jnp.float32),
            pltpu.VMEM_SHARED((N, HALF), jnp.float32),
        ],
        compiler_params=pltpu.CompilerParams(needs_layout_passes=False),
    )
    def _sc_scatter(h0, h1, srcE, dstE, p_in, zrows,
                    acc_out,
                    si_v, di_v, buf, pch, acc):
        c = lax.axis_index("c")
        s = lax.axis_index("s")

        @pl.when(s == 0)
        def _():
            pltpu.sync_copy(zrows, acc)

        plsc.subcore_barrier()

        def process(h_ref):
            def chunk(i, carry):
                row = s * NCH + i
                pltpu.sync_copy(srcE.at[row], si_v)
                pltpu.sync_copy(dstE.at[row], di_v)
                pltpu.sync_copy(
                    p_in.at[pl.ds(((c * NSUB + s) * NCH + i) * PW, PW)], pch)
                pltpu.sync_copy(h_ref.at[si_v.at[0]], buf)
                for g in range(K // 16):
                    p_list = [pch[pl.ds(hh * K + g * 16, 16)]
                              for hh in range(HPC)]
                    for j in range(16):
                        ej = g * 16 + j
                        for hh in range(HPC):
                            pj = p_list[hh][j]
                            sl0 = pl.ds(hh * 32, 16)
                            sl1 = pl.ds(hh * 32 + 16, 16)
                            buf[ej, sl0] = buf[ej, sl0] * pj
                            buf[ej, sl1] = buf[ej, sl1] * pj
                pltpu.sync_copy(buf, acc.at[di_v.at[0]], add=True)
                return carry

            lax.fori_loop(0, NCH, chunk, 0)

        @pl.when(c == 0)
        def _():
            process(h0)

        @pl.when(c == 1)
        def _():
            process(h1)

        plsc.subcore_barrier()

        @pl.when(s == 0)
        def _():
            pltpu.sync_copy(acc, acc_out.at[c])

    return _sc_scatter


def _tc_norm_body(a0_ref, a1_ref, dq_ref, b_ref, o_ref):
    a0 = a0_ref[...]
    a1 = a1_ref[...]
    dq = dq_ref[...]
    parts = []
    for half, a in enumerate((a0, a1)):
        for hh in range(HPC):
            g = half * HPC + hh
            num = a[:, hh * C:(hh + 1) * C]
            den = dq[:, g:g + 1]
            parts.append(num / (den + 1e-16))
    o_ref[...] = jnp.concatenate(parts, axis=1) + b_ref[...]


def _tc_norm(acc0, acc1, denq, bias2d):
    return pl.pallas_call(
        _tc_norm_body,
        grid=(N // BLK,),
        in_specs=[
            pl.BlockSpec((BLK, HALF), lambda i: (i, 0)),
            pl.BlockSpec((BLK, HALF), lambda i: (i, 0)),
            pl.BlockSpec((BLK, H), lambda i: (i, 0)),
            pl.BlockSpec((1, D), lambda i: (0, 0)),
        ],
        out_specs=pl.BlockSpec((BLK, D), lambda i: (i, 0)),
        out_shape=jax.ShapeDtypeStruct((N, D), jnp.float32),
    )(acc0, acc1, denq, bias2d)


def kernel(x, edge_index, W, att_src, att_dst, bias):
    eye = jnp.eye(H, dtype=jnp.float32)
    A_src = (att_src[:, :, None] * eye[:, None, :]).reshape(D, H)
    A_dst = (att_dst[:, :, None] * eye[:, None, :]).reshape(D, H)

    h0, h1, als, ald = _tc_project(x, W, A_src, A_dst)
    alsT = als.T.reshape(-1)
    aldT = ald.T.reshape(-1)

    srcE = edge_index[0].reshape(NSUB * NCH, 1, K)
    dstE = edge_index[1].reshape(NSUB * NCH, 1, K)
    zrows = jnp.zeros((N, HALF), jnp.float32)
    zflat = jnp.zeros((DTOT,), jnp.float32)

    p_all, den, _ = _get_sc_weights()(alsT, aldT, srcE, dstE, zflat)
    acc = _get_sc_scatter()(h0, h1, srcE, dstE, p_all, zrows)

    denq = den.reshape(H, DSTRIDE)[:, :N].T
    return _tc_norm(acc[0], acc[1], denq, bias.reshape(1, D))


# speedup vs baseline: 52.7140x; 1.6718x over previous
import functools

import jax
import jax.numpy as jnp
from jax import lax
from jax.experimental import pallas as pl
from jax.experimental.pallas import tpu as pltpu
from jax.experimental.pallas import tpu_sc as plsc

N = 10000
E = 320000
F = 128
H = 8
C = 32
D = H * C
HALF = D // 2
HPC = H // 2
NSUB = 16
K = 80
NCH = (E // NSUB) // K
PW = HPC * K
SA = 10
SB = 10
NSUP = NCH // SB
BLK = 1000
RSL = 2512
DSTRIDE = 10048
DTOT = HPC * DSTRIDE


def _tc_project_body(x_ref, w_ref, as_ref, ad_ref, h0_ref, h1_ref,
                     als_ref, ald_ref):
    h = jnp.dot(x_ref[...], w_ref[...], preferred_element_type=jnp.float32)
    h0_ref[...] = h[:, :HALF]
    h1_ref[...] = h[:, HALF:]
    als_ref[...] = jnp.dot(h, as_ref[...], preferred_element_type=jnp.float32)
    ald_ref[...] = jnp.dot(h, ad_ref[...], preferred_element_type=jnp.float32)


def _tc_project(x, W, A_src, A_dst):
    return pl.pallas_call(
        _tc_project_body,
        grid=(N // BLK,),
        in_specs=[
            pl.BlockSpec((BLK, F), lambda i: (i, 0)),
            pl.BlockSpec((F, D), lambda i: (0, 0)),
            pl.BlockSpec((D, H), lambda i: (0, 0)),
            pl.BlockSpec((D, H), lambda i: (0, 0)),
        ],
        out_specs=[
            pl.BlockSpec((BLK, HALF), lambda i: (i, 0)),
            pl.BlockSpec((BLK, HALF), lambda i: (i, 0)),
            pl.BlockSpec((BLK, H), lambda i: (i, 0)),
            pl.BlockSpec((BLK, H), lambda i: (i, 0)),
        ],
        out_shape=[
            jax.ShapeDtypeStruct((N, HALF), jnp.float32),
            jax.ShapeDtypeStruct((N, HALF), jnp.float32),
            jax.ShapeDtypeStruct((N, H), jnp.float32),
            jax.ShapeDtypeStruct((N, H), jnp.float32),
        ],
    )(x, W, A_src, A_dst)


def _sc_mesh():
    return plsc.VectorSubcoreMesh(core_axis_name="c", subcore_axis_name="s")


@functools.cache
def _get_sc_weights():
    @functools.partial(
        pl.kernel,
        out_type=[
            jax.ShapeDtypeStruct((2 * E * HPC,), jnp.float32),
            jax.ShapeDtypeStruct((2 * DTOT,), jnp.float32),
            jax.ShapeDtypeStruct((2 * NSUB * DTOT,), jnp.float32),
        ],
        mesh=_sc_mesh(),
        scratch_types=[
            pltpu.VMEM((HPC * N,), jnp.float32),
            pltpu.VMEM((HPC * N,), jnp.float32),
            pltpu.VMEM((SA * K,), jnp.int32),
            pltpu.VMEM((SA * K,), jnp.int32),
            pltpu.VMEM((SA * PW,), jnp.float32),
            pltpu.VMEM((DTOT,), jnp.float32),
        ],
        compiler_params=pltpu.CompilerParams(needs_layout_passes=False),
    )
    def _sc_weights(alsT, aldT, srcF, dstF, zflat,
                    p_out, den_out, den_part,
                    as_v, ad_v, si_v, di_v, pch, den_v):
        c = lax.axis_index("c")
        s = lax.axis_index("s")

        pltpu.sync_copy(alsT.at[pl.ds(c * (HPC * N), HPC * N)], as_v)
        pltpu.sync_copy(aldT.at[pl.ds(c * (HPC * N), HPC * N)], ad_v)
        pltpu.sync_copy(zflat, den_v)

        def chunk(i2, carry):
            for g in range(K // 16):
                s16 = si_v[pl.ds(i2 * K + g * 16, 16)]
                d16 = di_v[pl.ds(i2 * K + g * 16, 16)]
                for hh in range(HPC):
                    off = jnp.full((16,), hh * N, jnp.int32)
                    a_s = plsc.load_gather(as_v, [off + s16])
                    a_d = plsc.load_gather(ad_v, [off + d16])
                    z = a_s + a_d
                    e = jnp.maximum(z, 0.2 * z)
                    p = jnp.exp(e)
                    doff = jnp.full((16,), hh * DSTRIDE, jnp.int32)
                    plsc.addupdate_scatter(den_v, [doff + d16], p)
                    pch[pl.ds(i2 * PW + hh * K + g * 16, 16)] = p
            return carry

        def super_chunk(u, carry):
            tb = s * (NCH * K) + u * (SA * K)
            pltpu.sync_copy(srcF.at[pl.ds(tb, SA * K)], si_v)
            pltpu.sync_copy(dstF.at[pl.ds(tb, SA * K)], di_v)
            lax.fori_loop(0, SA, chunk, 0)
            pltpu.sync_copy(
                pch,
                p_out.at[pl.ds(((c * NSUB + s) * NCH + u * SA) * PW, SA * PW)])
            return carry

        lax.fori_loop(0, NCH // SA, super_chunk, 0)

        pltpu.sync_copy(den_v, den_part.at[pl.ds((c * NSUB + s) * DTOT, DTOT)])
        plsc.subcore_barrier()

        base = s * RSL
        pltpu.sync_copy(den_part.at[pl.ds(c * NSUB * DTOT + base, RSL)],
                        den_v.at[pl.ds(RSL, RSL)])

        def red(k, carry):
            pltpu.sync_copy(
                den_part.at[pl.ds((c * NSUB + k) * DTOT + base, RSL)],
                den_v.at[pl.ds(0, RSL)])
            for v in range(RSL // 16):
                sa = pl.ds(v * 16, 16)
                sb = pl.ds(RSL + v * 16, 16)
                den_v[sb] = den_v[sb] + den_v[sa]
            return carry

        lax.fori_loop(1, NSUB, red, 0)

        pltpu.sync_copy(den_v.at[pl.ds(RSL, RSL)],
                        den_out.at[pl.ds(c * DTOT + base, RSL)])

    return _sc_weights


@functools.cache
def _get_sc_scatter():
    @functools.partial(
        pl.kernel,
        out_type=jax.ShapeDtypeStruct((2, N, HALF), jnp.float32),
        mesh=_sc_mesh(),
        scratch_types=[
            pltpu.VMEM((SB, K), jnp.int32),
            pltpu.VMEM((SB, K), jnp.int32),
            pltpu.VMEM((K, HALF), jnp.float32),
            pltpu.VMEM((SB * PW,), jnp.float32),
            pltpu.VMEM_SHARED((N, HALF), jnp.float32),
        ],
        compiler_params=pltpu.CompilerParams(needs_layout_passes=False),
    )
    def _sc_scatter(h0, h1, srcSB, dstSB, p_in, zrows,
                    acc_out,
                    si_v, di_v, buf, pch, acc):
        c = lax.axis_index("c")
        s = lax.axis_index("s")

        @pl.when(s == 0)
        def _():
            pltpu.sync_copy(zrows, acc)

        plsc.subcore_barrier()

        def process(h_ref):
            def chunk(i2, carry2):
                pltpu.sync_copy(h_ref.at[si_v.at[i2]], buf)
                for g in range(K // 16):
                    p_list = [pch[pl.ds(i2 * PW + hh * K + g * 16, 16)]
                              for hh in range(HPC)]
                    for j in range(16):
                        ej = g * 16 + j
                        for hh in range(HPC):
                            pj = p_list[hh][j]
                            sl0 = pl.ds(hh * 32, 16)
                            sl1 = pl.ds(hh * 32 + 16, 16)
                            buf[ej, sl0] = buf[ej, sl0] * pj
                            buf[ej, sl1] = buf[ej, sl1] * pj
                pltpu.sync_copy(buf, acc.at[di_v.at[i2]], add=True)
                return carry2

            def super_chunk(u, carry):
                r = s * NSUP + u
                pltpu.sync_copy(srcSB.at[r], si_v)
                pltpu.sync_copy(dstSB.at[r], di_v)
                pltpu.sync_copy(
                    p_in.at[pl.ds(((c * NSUB + s) * NCH + u * SB) * PW,
                                  SB * PW)], pch)
                lax.fori_loop(0, SB, chunk, 0)
                return carry

            lax.fori_loop(0, NSUP, super_chunk, 0)

        @pl.when(c == 0)
        def _():
            process(h0)

        @pl.when(c == 1)
        def _():
            process(h1)

        plsc.subcore_barrier()

        @pl.when(s == 0)
        def _():
            pltpu.sync_copy(acc, acc_out.at[c])

    return _sc_scatter


def _tc_norm_body(a0_ref, a1_ref, dq_ref, b_ref, o_ref):
    a0 = a0_ref[...]
    a1 = a1_ref[...]
    dq = dq_ref[...]
    parts = []
    for half, a in enumerate((a0, a1)):
        for hh in range(HPC):
            g = half * HPC + hh
            num = a[:, hh * C:(hh + 1) * C]
            den = dq[:, g:g + 1]
            parts.append(num / (den + 1e-16))
    o_ref[...] = jnp.concatenate(parts, axis=1) + b_ref[...]


def _tc_norm(acc0, acc1, denq, bias2d):
    return pl.pallas_call(
        _tc_norm_body,
        grid=(N // BLK,),
        in_specs=[
            pl.BlockSpec((BLK, HALF), lambda i: (i, 0)),
            pl.BlockSpec((BLK, HALF), lambda i: (i, 0)),
            pl.BlockSpec((BLK, H), lambda i: (i, 0)),
            pl.BlockSpec((1, D), lambda i: (0, 0)),
        ],
        out_specs=pl.BlockSpec((BLK, D), lambda i: (i, 0)),
        out_shape=jax.ShapeDtypeStruct((N, D), jnp.float32),
    )(acc0, acc1, denq, bias2d)


def kernel(x, edge_index, W, att_src, att_dst, bias):
    eye = jnp.eye(H, dtype=jnp.float32)
    A_src = (att_src[:, :, None] * eye[:, None, :]).reshape(D, H)
    A_dst = (att_dst[:, :, None] * eye[:, None, :]).reshape(D, H)

    h0, h1, als, ald = _tc_project(x, W, A_src, A_dst)
    alsT = als.T.reshape(-1)
    aldT = ald.T.reshape(-1)

    srcF = edge_index[0]
    dstF = edge_index[1]
    srcSB = srcF.reshape(NSUB * NSUP, SB, K)
    dstSB = dstF.reshape(NSUB * NSUP, SB, K)
    zrows = jnp.zeros((N, HALF), jnp.float32)
    zflat = jnp.zeros((DTOT,), jnp.float32)

    p_all, den, _ = _get_sc_weights()(alsT, aldT, srcF, dstF, zflat)
    acc = _get_sc_scatter()(h0, h1, srcSB, dstSB, p_all, zrows)

    denq = den.reshape(H, DSTRIDE)[:, :N].T
    return _tc_norm(acc[0], acc[1], denq, bias.reshape(1, D))


# speedup vs baseline: 61.1922x; 1.1608x over previous
import functools

import jax
import jax.numpy as jnp
from jax import lax
from jax.experimental import pallas as pl
from jax.experimental.pallas import tpu as pltpu
from jax.experimental.pallas import tpu_sc as plsc

N = 10000
E = 320000
F = 128
H = 8
C = 32
D = H * C
HALF = D // 2
HPC = H // 2
NSUB = 16
K = 80
NCH = (E // NSUB) // K
PW = HPC * K
SA = 10
SB = 10
NSUP = NCH // SB
BLK = 1000
RSL = 2512
DSTRIDE = 10048
DTOT = HPC * DSTRIDE


def _tc_project_body(x_ref, w_ref, as_ref, ad_ref, h0_ref, h1_ref,
                     als_ref, ald_ref):
    h = jnp.dot(x_ref[...], w_ref[...], preferred_element_type=jnp.float32)
    h0_ref[...] = h[:, :HALF]
    h1_ref[...] = h[:, HALF:]
    als_ref[...] = jnp.dot(h, as_ref[...], preferred_element_type=jnp.float32)
    ald_ref[...] = jnp.dot(h, ad_ref[...], preferred_element_type=jnp.float32)


def _tc_project(x, W, A_src, A_dst):
    return pl.pallas_call(
        _tc_project_body,
        grid=(N // BLK,),
        in_specs=[
            pl.BlockSpec((BLK, F), lambda i: (i, 0)),
            pl.BlockSpec((F, D), lambda i: (0, 0)),
            pl.BlockSpec((D, H), lambda i: (0, 0)),
            pl.BlockSpec((D, H), lambda i: (0, 0)),
        ],
        out_specs=[
            pl.BlockSpec((BLK, HALF), lambda i: (i, 0)),
            pl.BlockSpec((BLK, HALF), lambda i: (i, 0)),
            pl.BlockSpec((BLK, H), lambda i: (i, 0)),
            pl.BlockSpec((BLK, H), lambda i: (i, 0)),
        ],
        out_shape=[
            jax.ShapeDtypeStruct((N, HALF), jnp.float32),
            jax.ShapeDtypeStruct((N, HALF), jnp.float32),
            jax.ShapeDtypeStruct((N, H), jnp.float32),
            jax.ShapeDtypeStruct((N, H), jnp.float32),
        ],
    )(x, W, A_src, A_dst)


def _sc_mesh():
    return plsc.VectorSubcoreMesh(core_axis_name="c", subcore_axis_name="s")


@functools.cache
def _get_sc_weights():
    @functools.partial(
        pl.kernel,
        out_type=[
            jax.ShapeDtypeStruct((2 * E * HPC,), jnp.float32),
            jax.ShapeDtypeStruct((2 * DTOT,), jnp.float32),
            jax.ShapeDtypeStruct((2 * NSUB * DTOT,), jnp.float32),
        ],
        mesh=_sc_mesh(),
        scratch_types=[
            pltpu.VMEM((HPC * N,), jnp.float32),
            pltpu.VMEM((HPC * N,), jnp.float32),
            pltpu.VMEM((SA * K,), jnp.int32),
            pltpu.VMEM((SA * K,), jnp.int32),
            pltpu.VMEM((SA * PW,), jnp.float32),
            pltpu.VMEM((DTOT,), jnp.float32),
        ],
        compiler_params=pltpu.CompilerParams(needs_layout_passes=False),
    )
    def _sc_weights(alsT, aldT, srcF, dstF, zflat,
                    p_out, den_out, den_part,
                    as_v, ad_v, si_v, di_v, pch, den_v):
        c = lax.axis_index("c")
        s = lax.axis_index("s")

        pltpu.sync_copy(alsT.at[pl.ds(c * (HPC * N), HPC * N)], as_v)
        pltpu.sync_copy(aldT.at[pl.ds(c * (HPC * N), HPC * N)], ad_v)
        pltpu.sync_copy(zflat, den_v)

        def chunk(i2, carry):
            for g in range(K // 16):
                s16 = si_v[pl.ds(i2 * K + g * 16, 16)]
                d16 = di_v[pl.ds(i2 * K + g * 16, 16)]
                for hh in range(HPC):
                    off = jnp.full((16,), hh * N, jnp.int32)
                    a_s = plsc.load_gather(as_v, [off + s16])
                    a_d = plsc.load_gather(ad_v, [off + d16])
                    z = a_s + a_d
                    e = jnp.maximum(z, 0.2 * z)
                    p = jnp.exp(e)
                    doff = jnp.full((16,), hh * DSTRIDE, jnp.int32)
                    plsc.addupdate_scatter(den_v, [doff + d16], p)
                    pch[pl.ds(i2 * PW + hh * K + g * 16, 16)] = p
            return carry

        def super_chunk(u, carry):
            tb = s * (NCH * K) + u * (SA * K)
            pltpu.sync_copy(srcF.at[pl.ds(tb, SA * K)], si_v)
            pltpu.sync_copy(dstF.at[pl.ds(tb, SA * K)], di_v)
            lax.fori_loop(0, SA, chunk, 0)
            pltpu.sync_copy(
                pch,
                p_out.at[pl.ds(((c * NSUB + s) * NCH + u * SA) * PW, SA * PW)])
            return carry

        lax.fori_loop(0, NCH // SA, super_chunk, 0)

        pltpu.sync_copy(den_v, den_part.at[pl.ds((c * NSUB + s) * DTOT, DTOT)])
        plsc.subcore_barrier()

        base = s * RSL
        pltpu.sync_copy(den_part.at[pl.ds(c * NSUB * DTOT + base, RSL)],
                        den_v.at[pl.ds(RSL, RSL)])

        def red(k, carry):
            pltpu.sync_copy(
                den_part.at[pl.ds((c * NSUB + k) * DTOT + base, RSL)],
                den_v.at[pl.ds(0, RSL)])
            for v in range(RSL // 16):
                sa = pl.ds(v * 16, 16)
                sb = pl.ds(RSL + v * 16, 16)
                den_v[sb] = den_v[sb] + den_v[sa]
            return carry

        lax.fori_loop(1, NSUB, red, 0)

        pltpu.sync_copy(den_v.at[pl.ds(RSL, RSL)],
                        den_out.at[pl.ds(c * DTOT + base, RSL)])

    return _sc_weights


@functools.cache
def _get_sc_scatter():
    @functools.partial(
        pl.kernel,
        out_type=jax.ShapeDtypeStruct((2, N, HALF), jnp.float32),
        mesh=_sc_mesh(),
        scratch_types=[
            pltpu.VMEM((SB, K), jnp.int32),
            pltpu.VMEM((SB, K), jnp.int32),
            pltpu.VMEM((K, HALF), jnp.float32),
            pltpu.VMEM((K, HALF), jnp.float32),
            pltpu.VMEM((SB * PW,), jnp.float32),
            pltpu.VMEM_SHARED((N, HALF), jnp.float32),
            pltpu.SemaphoreType.DMA,
            pltpu.SemaphoreType.DMA,
        ],
        compiler_params=pltpu.CompilerParams(needs_layout_passes=False),
    )
    def _sc_scatter(h0, h1, srcSB, dstSB, p_in, zrows,
                    acc_out,
                    si_v, di_v, buf0, buf1, pch, acc, sem_g, sem_s):
        c = lax.axis_index("c")
        s = lax.axis_index("s")

        @pl.when(s == 0)
        def _():
            pltpu.sync_copy(zrows, acc)

        plsc.subcore_barrier()

        def process(h_ref):
            bufs = (buf0, buf1)

            def do_chunk(t, b, other):
                pltpu.make_async_copy(
                    h_ref.at[si_v.at[0]], bufs[b], sem_g).wait()

                @pl.when(t >= 1)
                def _():
                    pltpu.make_async_copy(
                        bufs[other], acc.at[di_v.at[0]], sem_s).wait()

                @pl.when(t < SB - 1)
                def _():
                    pltpu.async_copy(
                        h_ref.at[si_v.at[t + 1]], bufs[other], sem_g)

                for g in range(K // 16):
                    p_list = [pch[pl.ds(t * PW + hh * K + g * 16, 16)]
                              for hh in range(HPC)]
                    for j in range(16):
                        ej = g * 16 + j
                        for hh in range(HPC):
                            pj = p_list[hh][j]
                            sl0 = pl.ds(hh * 32, 16)
                            sl1 = pl.ds(hh * 32 + 16, 16)
                            bufs[b][ej, sl0] = bufs[b][ej, sl0] * pj
                            bufs[b][ej, sl1] = bufs[b][ej, sl1] * pj

                pltpu.async_copy(bufs[b], acc.at[di_v.at[t]], sem_s, add=True)

            def chunk_pair(t2, carry2):
                do_chunk(2 * t2, 0, 1)
                do_chunk(2 * t2 + 1, 1, 0)
                return carry2

            def super_chunk(u, carry):
                r = s * NSUP + u
                pltpu.sync_copy(srcSB.at[r], si_v)
                pltpu.sync_copy(dstSB.at[r], di_v)
                pltpu.sync_copy(
                    p_in.at[pl.ds(((c * NSUB + s) * NCH + u * SB) * PW,
                                  SB * PW)], pch)
                pltpu.async_copy(h_ref.at[si_v.at[0]], buf0, sem_g)
                lax.fori_loop(0, SB // 2, chunk_pair, 0)
                pltpu.make_async_copy(
                    buf1, acc.at[di_v.at[0]], sem_s).wait()
                return carry

            lax.fori_loop(0, NSUP, super_chunk, 0)

        @pl.when(c == 0)
        def _():
            process(h0)

        @pl.when(c == 1)
        def _():
            process(h1)

        plsc.subcore_barrier()

        @pl.when(s == 0)
        def _():
            pltpu.sync_copy(acc, acc_out.at[c])

    return _sc_scatter


def _tc_norm_body(a0_ref, a1_ref, dq_ref, b_ref, o_ref):
    a0 = a0_ref[...]
    a1 = a1_ref[...]
    dq = dq_ref[...]
    parts = []
    for half, a in enumerate((a0, a1)):
        for hh in range(HPC):
            g = half * HPC + hh
            num = a[:, hh * C:(hh + 1) * C]
            den = dq[:, g:g + 1]
            parts.append(num / (den + 1e-16))
    o_ref[...] = jnp.concatenate(parts, axis=1) + b_ref[...]


def _tc_norm(acc0, acc1, denq, bias2d):
    return pl.pallas_call(
        _tc_norm_body,
        grid=(N // BLK,),
        in_specs=[
            pl.BlockSpec((BLK, HALF), lambda i: (i, 0)),
            pl.BlockSpec((BLK, HALF), lambda i: (i, 0)),
            pl.BlockSpec((BLK, H), lambda i: (i, 0)),
            pl.BlockSpec((1, D), lambda i: (0, 0)),
        ],
        out_specs=pl.BlockSpec((BLK, D), lambda i: (i, 0)),
        out_shape=jax.ShapeDtypeStruct((N, D), jnp.float32),
    )(acc0, acc1, denq, bias2d)


def kernel(x, edge_index, W, att_src, att_dst, bias):
    eye = jnp.eye(H, dtype=jnp.float32)
    A_src = (att_src[:, :, None] * eye[:, None, :]).reshape(D, H)
    A_dst = (att_dst[:, :, None] * eye[:, None, :]).reshape(D, H)

    h0, h1, als, ald = _tc_project(x, W, A_src, A_dst)
    alsT = als.T.reshape(-1)
    aldT = ald.T.reshape(-1)

    srcF = edge_index[0]
    dstF = edge_index[1]
    srcSB = srcF.reshape(NSUB * NSUP, SB, K)
    dstSB = dstF.reshape(NSUB * NSUP, SB, K)
    zrows = jnp.zeros((N, HALF), jnp.float32)
    zflat = jnp.zeros((DTOT,), jnp.float32)

    p_all, den, _ = _get_sc_weights()(alsT, aldT, srcF, dstF, zflat)
    acc = _get_sc_scatter()(h0, h1, srcSB, dstSB, p_all, zrows)

    denq = den.reshape(H, DSTRIDE)[:, :N].T
    return _tc_norm(acc[0], acc[1], denq, bias.reshape(1, D))


# speedup vs baseline: 68.1807x; 1.1142x over previous
import functools

import jax
import jax.numpy as jnp
from jax import lax
from jax.experimental import pallas as pl
from jax.experimental.pallas import tpu as pltpu
from jax.experimental.pallas import tpu_sc as plsc

N = 10000
E = 320000
F = 128
H = 8
C = 32
D = H * C
HALF = D // 2
HPC = H // 2
NSUB = 16
K = 80
NCH = (E // NSUB) // K
PW = HPC * K
SA = 10
SB = 10
NSUP = NCH // SB
BLK = 1000
RSL = 2512
DSTRIDE = 10048
DTOT = HPC * DSTRIDE


def _tc_project_body(x_ref, w_ref, as_ref, ad_ref, h0_ref, h1_ref,
                     als_ref, ald_ref):
    h = jnp.dot(x_ref[...], w_ref[...], preferred_element_type=jnp.float32)
    h0_ref[...] = h[:, :HALF]
    h1_ref[...] = h[:, HALF:]
    als_ref[...] = jnp.dot(h, as_ref[...], preferred_element_type=jnp.float32)
    ald_ref[...] = jnp.dot(h, ad_ref[...], preferred_element_type=jnp.float32)


def _tc_project(x, W, A_src, A_dst):
    return pl.pallas_call(
        _tc_project_body,
        grid=(N // BLK,),
        in_specs=[
            pl.BlockSpec((BLK, F), lambda i: (i, 0)),
            pl.BlockSpec((F, D), lambda i: (0, 0)),
            pl.BlockSpec((D, H), lambda i: (0, 0)),
            pl.BlockSpec((D, H), lambda i: (0, 0)),
        ],
        out_specs=[
            pl.BlockSpec((BLK, HALF), lambda i: (i, 0)),
            pl.BlockSpec((BLK, HALF), lambda i: (i, 0)),
            pl.BlockSpec((BLK, H), lambda i: (i, 0)),
            pl.BlockSpec((BLK, H), lambda i: (i, 0)),
        ],
        out_shape=[
            jax.ShapeDtypeStruct((N, HALF), jnp.float32),
            jax.ShapeDtypeStruct((N, HALF), jnp.float32),
            jax.ShapeDtypeStruct((N, H), jnp.float32),
            jax.ShapeDtypeStruct((N, H), jnp.float32),
        ],
    )(x, W, A_src, A_dst)


def _sc_mesh():
    return plsc.VectorSubcoreMesh(core_axis_name="c", subcore_axis_name="s")


@functools.cache
def _get_sc_weights():
    @functools.partial(
        pl.kernel,
        out_type=[
            jax.ShapeDtypeStruct((2 * E * HPC,), jnp.float32),
            jax.ShapeDtypeStruct((2 * DTOT,), jnp.float32),
            jax.ShapeDtypeStruct((2 * NSUB * DTOT,), jnp.float32),
        ],
        mesh=_sc_mesh(),
        scratch_types=[
            pltpu.VMEM((HPC * N,), jnp.float32),
            pltpu.VMEM((HPC * N,), jnp.float32),
            pltpu.VMEM((SA * K,), jnp.int32),
            pltpu.VMEM((SA * K,), jnp.int32),
            pltpu.VMEM((SA * PW,), jnp.float32),
            pltpu.VMEM((DTOT,), jnp.float32),
        ],
        compiler_params=pltpu.CompilerParams(needs_layout_passes=False),
    )
    def _sc_weights(alsT, aldT, srcF, dstF, zflat,
                    p_out, den_out, den_part,
                    as_v, ad_v, si_v, di_v, pch, den_v):
        c = lax.axis_index("c")
        s = lax.axis_index("s")

        pltpu.sync_copy(alsT.at[pl.ds(c * (HPC * N), HPC * N)], as_v)
        pltpu.sync_copy(aldT.at[pl.ds(c * (HPC * N), HPC * N)], ad_v)
        pltpu.sync_copy(zflat, den_v)

        def chunk(i2, carry):
            for g in range(K // 16):
                s16 = si_v[pl.ds(i2 * K + g * 16, 16)]
                d16 = di_v[pl.ds(i2 * K + g * 16, 16)]
                for hh in range(HPC):
                    off = jnp.full((16,), hh * N, jnp.int32)
                    a_s = plsc.load_gather(as_v, [off + s16])
                    a_d = plsc.load_gather(ad_v, [off + d16])
                    z = a_s + a_d
                    e = jnp.maximum(z, 0.2 * z)
                    p = jnp.exp(e)
                    doff = jnp.full((16,), hh * DSTRIDE, jnp.int32)
                    plsc.addupdate_scatter(den_v, [doff + d16], p)
                    pch[pl.ds(i2 * PW + hh * K + g * 16, 16)] = p
            return carry

        def super_chunk(u, carry):
            tb = s * (NCH * K) + u * (SA * K)
            pltpu.sync_copy(srcF.at[pl.ds(tb, SA * K)], si_v)
            pltpu.sync_copy(dstF.at[pl.ds(tb, SA * K)], di_v)
            lax.fori_loop(0, SA, chunk, 0)
            pltpu.sync_copy(
                pch,
                p_out.at[pl.ds(((c * NSUB + s) * NCH + u * SA) * PW, SA * PW)])
            return carry

        lax.fori_loop(0, NCH // SA, super_chunk, 0)

        pltpu.sync_copy(den_v, den_part.at[pl.ds((c * NSUB + s) * DTOT, DTOT)])
        plsc.subcore_barrier()

        base = s * RSL
        pltpu.sync_copy(den_part.at[pl.ds(c * NSUB * DTOT + base, RSL)],
                        den_v.at[pl.ds(RSL, RSL)])

        def red(k, carry):
            pltpu.sync_copy(
                den_part.at[pl.ds((c * NSUB + k) * DTOT + base, RSL)],
                den_v.at[pl.ds(0, RSL)])
            for v in range(RSL // 16):
                sa = pl.ds(v * 16, 16)
                sb = pl.ds(RSL + v * 16, 16)
                den_v[sb] = den_v[sb] + den_v[sa]
            return carry

        lax.fori_loop(1, NSUB, red, 0)

        pltpu.sync_copy(den_v.at[pl.ds(RSL, RSL)],
                        den_out.at[pl.ds(c * DTOT + base, RSL)])

    return _sc_weights


@functools.cache
def _get_sc_scatter():
    @functools.partial(
        pl.kernel,
        out_type=jax.ShapeDtypeStruct((2, N, HALF), jnp.float32),
        mesh=_sc_mesh(),
        scratch_types=[
            pltpu.VMEM((SB, K), jnp.int32),
            pltpu.VMEM((SB, K), jnp.int32),
            pltpu.VMEM((K, HALF), jnp.float32),
            pltpu.VMEM((K, HALF), jnp.float32),
            pltpu.VMEM((K, HALF), jnp.float32),
            pltpu.VMEM((K, HALF), jnp.float32),
            pltpu.VMEM((SB * PW,), jnp.float32),
            pltpu.VMEM_SHARED((N, HALF), jnp.float32),
            pltpu.SemaphoreType.DMA,
            pltpu.SemaphoreType.DMA,
        ],
        compiler_params=pltpu.CompilerParams(needs_layout_passes=False),
    )
    def _sc_scatter(h0, h1, srcSB, dstSB, p_in, zrows,
                    acc_out,
                    si_v, di_v, gb0, gb1, sb0, sb1, pch, acc, sem_g, sem_s):
        c = lax.axis_index("c")
        s = lax.axis_index("s")

        @pl.when(s == 0)
        def _():
            pltpu.sync_copy(zrows, acc)

        plsc.subcore_barrier()

        def process(h_ref):
            gbufs = (gb0, gb1)
            sbufs = (sb0, sb1)

            def do_chunk(t, b, other):
                pltpu.make_async_copy(
                    h_ref.at[si_v.at[0]], gbufs[b], sem_g).wait()

                @pl.when(t < SB - 1)
                def _():
                    pltpu.async_copy(
                        h_ref.at[si_v.at[t + 1]], gbufs[other], sem_g)

                @pl.when(t >= 2)
                def _():
                    pltpu.make_async_copy(
                        sbufs[b], acc.at[di_v.at[0]], sem_s).wait()

                for g in range(K // 16):
                    p_list = [pch[pl.ds(t * PW + hh * K + g * 16, 16)]
                              for hh in range(HPC)]
                    for j in range(16):
                        ej = g * 16 + j
                        for hh in range(HPC):
                            pj = p_list[hh][j]
                            sl0 = pl.ds(hh * 32, 16)
                            sl1 = pl.ds(hh * 32 + 16, 16)
                            sbufs[b][ej, sl0] = gbufs[b][ej, sl0] * pj
                            sbufs[b][ej, sl1] = gbufs[b][ej, sl1] * pj

                pltpu.async_copy(sbufs[b], acc.at[di_v.at[t]], sem_s, add=True)

            def chunk_pair(t2, carry2):
                do_chunk(2 * t2, 0, 1)
                do_chunk(2 * t2 + 1, 1, 0)
                return carry2

            def super_chunk(u, carry):
                r = s * NSUP + u
                pltpu.sync_copy(srcSB.at[r], si_v)
                pltpu.sync_copy(dstSB.at[r], di_v)
                pltpu.sync_copy(
                    p_in.at[pl.ds(((c * NSUB + s) * NCH + u * SB) * PW,
                                  SB * PW)], pch)
                pltpu.async_copy(h_ref.at[si_v.at[0]], gb0, sem_g)
                lax.fori_loop(0, SB // 2, chunk_pair, 0)
                pltpu.make_async_copy(
                    sb0, acc.at[di_v.at[0]], sem_s).wait()
                pltpu.make_async_copy(
                    sb1, acc.at[di_v.at[0]], sem_s).wait()
                return carry

            lax.fori_loop(0, NSUP, super_chunk, 0)

        @pl.when(c == 0)
        def _():
            process(h0)

        @pl.when(c == 1)
        def _():
            process(h1)

        plsc.subcore_barrier()

        @pl.when(s == 0)
        def _():
            pltpu.sync_copy(acc, acc_out.at[c])

    return _sc_scatter


def _tc_norm_body(a0_ref, a1_ref, dq_ref, b_ref, o_ref):
    a0 = a0_ref[...]
    a1 = a1_ref[...]
    dq = dq_ref[...]
    parts = []
    for half, a in enumerate((a0, a1)):
        for hh in range(HPC):
            g = half * HPC + hh
            num = a[:, hh * C:(hh + 1) * C]
            den = dq[:, g:g + 1]
            parts.append(num / (den + 1e-16))
    o_ref[...] = jnp.concatenate(parts, axis=1) + b_ref[...]


def _tc_norm(acc0, acc1, denq, bias2d):
    return pl.pallas_call(
        _tc_norm_body,
        grid=(N // BLK,),
        in_specs=[
            pl.BlockSpec((BLK, HALF), lambda i: (i, 0)),
            pl.BlockSpec((BLK, HALF), lambda i: (i, 0)),
            pl.BlockSpec((BLK, H), lambda i: (i, 0)),
            pl.BlockSpec((1, D), lambda i: (0, 0)),
        ],
        out_specs=pl.BlockSpec((BLK, D), lambda i: (i, 0)),
        out_shape=jax.ShapeDtypeStruct((N, D), jnp.float32),
    )(acc0, acc1, denq, bias2d)


def kernel(x, edge_index, W, att_src, att_dst, bias):
    eye = jnp.eye(H, dtype=jnp.float32)
    A_src = (att_src[:, :, None] * eye[:, None, :]).reshape(D, H)
    A_dst = (att_dst[:, :, None] * eye[:, None, :]).reshape(D, H)

    h0, h1, als, ald = _tc_project(x, W, A_src, A_dst)
    alsT = als.T.reshape(-1)
    aldT = ald.T.reshape(-1)

    srcF = edge_index[0]
    dstF = edge_index[1]
    srcSB = srcF.reshape(NSUB * NSUP, SB, K)
    dstSB = dstF.reshape(NSUB * NSUP, SB, K)
    zrows = jnp.zeros((N, HALF), jnp.float32)
    zflat = jnp.zeros((DTOT,), jnp.float32)

    p_all, den, _ = _get_sc_weights()(alsT, aldT, srcF, dstF, zflat)
    acc = _get_sc_scatter()(h0, h1, srcSB, dstSB, p_all, zrows)

    denq = den.reshape(H, DSTRIDE)[:, :N].T
    return _tc_norm(acc[0], acc[1], denq, bias.reshape(1, D))


# speedup vs baseline: 71.7760x; 1.0527x over previous
import functools

import jax
import jax.numpy as jnp
from jax import lax
from jax.experimental import pallas as pl
from jax.experimental.pallas import tpu as pltpu
from jax.experimental.pallas import tpu_sc as plsc

N = 10000
E = 320000
F = 128
H = 8
C = 32
D = H * C
HALF = D // 2
HPC = H // 2
NSUB = 16
K = 80
NCH = (E // NSUB) // K
PW = HPC * K
SA = 5
NSUPA = NCH // SA
SB = 10
NSUP = NCH // SB
BLK = 1000
RSL = 2512
DSTRIDE = 10048
DTOT = HPC * DSTRIDE


def _tc_project_body(x_ref, w_ref, as_ref, ad_ref, h0_ref, h1_ref,
                     als_ref, ald_ref):
    h = jnp.dot(x_ref[...], w_ref[...], preferred_element_type=jnp.float32)
    h0_ref[...] = h[:, :HALF]
    h1_ref[...] = h[:, HALF:]
    als_ref[...] = jnp.dot(h, as_ref[...], preferred_element_type=jnp.float32)
    ald_ref[...] = jnp.dot(h, ad_ref[...], preferred_element_type=jnp.float32)


def _tc_project(x, W, A_src, A_dst):
    return pl.pallas_call(
        _tc_project_body,
        grid=(N // BLK,),
        in_specs=[
            pl.BlockSpec((BLK, F), lambda i: (i, 0)),
            pl.BlockSpec((F, D), lambda i: (0, 0)),
            pl.BlockSpec((D, H), lambda i: (0, 0)),
            pl.BlockSpec((D, H), lambda i: (0, 0)),
        ],
        out_specs=[
            pl.BlockSpec((BLK, HALF), lambda i: (i, 0)),
            pl.BlockSpec((BLK, HALF), lambda i: (i, 0)),
            pl.BlockSpec((BLK, H), lambda i: (i, 0)),
            pl.BlockSpec((BLK, H), lambda i: (i, 0)),
        ],
        out_shape=[
            jax.ShapeDtypeStruct((N, HALF), jnp.float32),
            jax.ShapeDtypeStruct((N, HALF), jnp.float32),
            jax.ShapeDtypeStruct((N, H), jnp.float32),
            jax.ShapeDtypeStruct((N, H), jnp.float32),
        ],
    )(x, W, A_src, A_dst)


def _sc_mesh():
    return plsc.VectorSubcoreMesh(core_axis_name="c", subcore_axis_name="s")


@functools.cache
def _get_sc_weights():
    @functools.partial(
        pl.kernel,
        out_type=[
            jax.ShapeDtypeStruct((2 * E * HPC,), jnp.float32),
            jax.ShapeDtypeStruct((2 * DTOT,), jnp.float32),
            jax.ShapeDtypeStruct((2 * NSUB * DTOT,), jnp.float32),
        ],
        mesh=_sc_mesh(),
        scratch_types=[
            pltpu.VMEM((HPC * N,), jnp.float32),
            pltpu.VMEM((HPC * N,), jnp.float32),
            pltpu.VMEM((SA * K,), jnp.int32),
            pltpu.VMEM((SA * K,), jnp.int32),
            pltpu.VMEM((SA * K,), jnp.int32),
            pltpu.VMEM((SA * K,), jnp.int32),
            pltpu.VMEM((SA * PW,), jnp.float32),
            pltpu.VMEM((SA * PW,), jnp.float32),
            pltpu.VMEM((DTOT,), jnp.float32),
            pltpu.SemaphoreType.DMA,
            pltpu.SemaphoreType.DMA,
        ],
        compiler_params=pltpu.CompilerParams(needs_layout_passes=False),
    )
    def _sc_weights(alsT, aldT, srcF, dstF, zflat,
                    p_out, den_out, den_part,
                    as_v, ad_v, si0, di0, si1, di1, pc0, pc1, den_v,
                    sem_i, sem_p):
        c = lax.axis_index("c")
        s = lax.axis_index("s")

        pltpu.sync_copy(alsT.at[pl.ds(c * (HPC * N), HPC * N)], as_v)
        pltpu.sync_copy(aldT.at[pl.ds(c * (HPC * N), HPC * N)], ad_v)
        pltpu.sync_copy(zflat, den_v)

        sis = (si0, si1)
        dis = (di0, di1)
        pcs = (pc0, pc1)
        tbase = s * (NCH * K)
        pbase = (c * NSUB + s) * NCH * PW

        def issue_idx(u, b):
            tb = tbase + u * (SA * K)
            pltpu.async_copy(srcF.at[pl.ds(tb, SA * K)], sis[b], sem_i)
            pltpu.async_copy(dstF.at[pl.ds(tb, SA * K)], dis[b], sem_i)

        def chunk_maker(si_v, di_v, pch):
            def chunk(i2, carry):
                for g in range(K // 16):
                    s16 = si_v[pl.ds(i2 * K + g * 16, 16)]
                    d16 = di_v[pl.ds(i2 * K + g * 16, 16)]
                    for hh in range(HPC):
                        off = jnp.full((16,), hh * N, jnp.int32)
                        a_s = plsc.load_gather(as_v, [off + s16])
                        a_d = plsc.load_gather(ad_v, [off + d16])
                        z = a_s + a_d
                        e = jnp.maximum(z, 0.2 * z)
                        p = jnp.exp(e)
                        doff = jnp.full((16,), hh * DSTRIDE, jnp.int32)
                        plsc.addupdate_scatter(den_v, [doff + d16], p)
                        pch[pl.ds(i2 * PW + hh * K + g * 16, 16)] = p
                return carry
            return chunk

        def do_super(u, b, other):
            pltpu.make_async_copy(
                srcF.at[pl.ds(0, SA * K)], sis[b], sem_i).wait()
            pltpu.make_async_copy(
                dstF.at[pl.ds(0, SA * K)], dis[b], sem_i).wait()

            @pl.when(u < NSUPA - 1)
            def _():
                issue_idx(u + 1, other)

            @pl.when(u >= 2)
            def _():
                pltpu.make_async_copy(
                    pcs[b], p_out.at[pl.ds(0, SA * PW)], sem_p).wait()

            lax.fori_loop(0, SA, chunk_maker(sis[b], dis[b], pcs[b]), 0)
            pltpu.async_copy(
                pcs[b], p_out.at[pl.ds(pbase + u * (SA * PW), SA * PW)], sem_p)

        def super_pair(u2, carry):
            do_super(2 * u2, 0, 1)
            do_super(2 * u2 + 1, 1, 0)
            return carry

        issue_idx(0, 0)
        lax.fori_loop(0, NSUPA // 2, super_pair, 0)
        pltpu.make_async_copy(pc0, p_out.at[pl.ds(0, SA * PW)], sem_p).wait()
        pltpu.make_async_copy(pc1, p_out.at[pl.ds(0, SA * PW)], sem_p).wait()

        pltpu.sync_copy(den_v, den_part.at[pl.ds((c * NSUB + s) * DTOT, DTOT)])
        plsc.subcore_barrier()

        base = s * RSL
        pltpu.sync_copy(den_part.at[pl.ds(c * NSUB * DTOT + base, RSL)],
                        den_v.at[pl.ds(RSL, RSL)])

        def red(k, carry):
            pltpu.sync_copy(
                den_part.at[pl.ds((c * NSUB + k) * DTOT + base, RSL)],
                den_v.at[pl.ds(0, RSL)])
            for v in range(RSL // 16):
                sa = pl.ds(v * 16, 16)
                sb = pl.ds(RSL + v * 16, 16)
                den_v[sb] = den_v[sb] + den_v[sa]
            return carry

        lax.fori_loop(1, NSUB, red, 0)

        pltpu.sync_copy(den_v.at[pl.ds(RSL, RSL)],
                        den_out.at[pl.ds(c * DTOT + base, RSL)])

    return _sc_weights


@functools.cache
def _get_sc_scatter():
    @functools.partial(
        pl.kernel,
        out_type=jax.ShapeDtypeStruct((2, N, HALF), jnp.float32),
        mesh=_sc_mesh(),
        scratch_types=[
            pltpu.VMEM((SB, K), jnp.int32),
            pltpu.VMEM((SB, K), jnp.int32),
            pltpu.VMEM((K, HALF), jnp.float32),
            pltpu.VMEM((K, HALF), jnp.float32),
            pltpu.VMEM((K, HALF), jnp.float32),
            pltpu.VMEM((K, HALF), jnp.float32),
            pltpu.VMEM((SB * PW,), jnp.float32),
            pltpu.VMEM_SHARED((N, HALF), jnp.float32),
            pltpu.SemaphoreType.DMA,
            pltpu.SemaphoreType.DMA,
        ],
        compiler_params=pltpu.CompilerParams(needs_layout_passes=False),
    )
    def _sc_scatter(h0, h1, srcSB, dstSB, p_in, zrows,
                    acc_out,
                    si_v, di_v, gb0, gb1, sb0, sb1, pch, acc, sem_g, sem_s):
        c = lax.axis_index("c")
        s = lax.axis_index("s")

        @pl.when(s == 0)
        def _():
            pltpu.sync_copy(zrows, acc)

        plsc.subcore_barrier()

        def process(h_ref):
            gbufs = (gb0, gb1)
            sbufs = (sb0, sb1)

            def do_chunk(t, b, other):
                pltpu.make_async_copy(
                    h_ref.at[si_v.at[0]], gbufs[b], sem_g).wait()

                @pl.when(t < SB - 1)
                def _():
                    pltpu.async_copy(
                        h_ref.at[si_v.at[t + 1]], gbufs[other], sem_g)

                @pl.when(t >= 2)
                def _():
                    pltpu.make_async_copy(
                        sbufs[b], acc.at[di_v.at[0]], sem_s).wait()

                for g in range(K // 16):
                    p_list = [pch[pl.ds(t * PW + hh * K + g * 16, 16)]
                              for hh in range(HPC)]
                    for j in range(16):
                        ej = g * 16 + j
                        for hh in range(HPC):
                            pj = p_list[hh][j]
                            sl0 = pl.ds(hh * 32, 16)
                            sl1 = pl.ds(hh * 32 + 16, 16)
                            sbufs[b][ej, sl0] = gbufs[b][ej, sl0] * pj
                            sbufs[b][ej, sl1] = gbufs[b][ej, sl1] * pj

                pltpu.async_copy(sbufs[b], acc.at[di_v.at[t]], sem_s, add=True)

            def chunk_pair(t2, carry2):
                do_chunk(2 * t2, 0, 1)
                do_chunk(2 * t2 + 1, 1, 0)
                return carry2

            def super_chunk(u, carry):
                r = s * NSUP + u
                pltpu.sync_copy(srcSB.at[r], si_v)
                pltpu.sync_copy(dstSB.at[r], di_v)
                pltpu.sync_copy(
                    p_in.at[pl.ds(((c * NSUB + s) * NCH + u * SB) * PW,
                                  SB * PW)], pch)
                pltpu.async_copy(h_ref.at[si_v.at[0]], gb0, sem_g)
                lax.fori_loop(0, SB // 2, chunk_pair, 0)
                pltpu.make_async_copy(
                    sb0, acc.at[di_v.at[0]], sem_s).wait()
                pltpu.make_async_copy(
                    sb1, acc.at[di_v.at[0]], sem_s).wait()
                return carry

            lax.fori_loop(0, NSUP, super_chunk, 0)

        @pl.when(c == 0)
        def _():
            process(h0)

        @pl.when(c == 1)
        def _():
            process(h1)

        plsc.subcore_barrier()

        @pl.when(s == 0)
        def _():
            pltpu.sync_copy(acc, acc_out.at[c])

    return _sc_scatter


def _tc_norm_body(a0_ref, a1_ref, dq_ref, b_ref, o_ref):
    a0 = a0_ref[...]
    a1 = a1_ref[...]
    dq = dq_ref[...]
    parts = []
    for half, a in enumerate((a0, a1)):
        for hh in range(HPC):
            g = half * HPC + hh
            num = a[:, hh * C:(hh + 1) * C]
            den = dq[:, g:g + 1]
            parts.append(num / (den + 1e-16))
    o_ref[...] = jnp.concatenate(parts, axis=1) + b_ref[...]


def _tc_norm(acc0, acc1, denq, bias2d):
    return pl.pallas_call(
        _tc_norm_body,
        grid=(N // BLK,),
        in_specs=[
            pl.BlockSpec((BLK, HALF), lambda i: (i, 0)),
            pl.BlockSpec((BLK, HALF), lambda i: (i, 0)),
            pl.BlockSpec((BLK, H), lambda i: (i, 0)),
            pl.BlockSpec((1, D), lambda i: (0, 0)),
        ],
        out_specs=pl.BlockSpec((BLK, D), lambda i: (i, 0)),
        out_shape=jax.ShapeDtypeStruct((N, D), jnp.float32),
    )(acc0, acc1, denq, bias2d)


def kernel(x, edge_index, W, att_src, att_dst, bias):
    eye = jnp.eye(H, dtype=jnp.float32)
    A_src = (att_src[:, :, None] * eye[:, None, :]).reshape(D, H)
    A_dst = (att_dst[:, :, None] * eye[:, None, :]).reshape(D, H)

    h0, h1, als, ald = _tc_project(x, W, A_src, A_dst)
    alsT = als.T.reshape(-1)
    aldT = ald.T.reshape(-1)

    srcF = edge_index[0]
    dstF = edge_index[1]
    srcSB = srcF.reshape(NSUB * NSUP, SB, K)
    dstSB = dstF.reshape(NSUB * NSUP, SB, K)
    zrows = jnp.zeros((N, HALF), jnp.float32)
    zflat = jnp.zeros((DTOT,), jnp.float32)

    p_all, den, _ = _get_sc_weights()(alsT, aldT, srcF, dstF, zflat)
    acc = _get_sc_scatter()(h0, h1, srcSB, dstSB, p_all, zrows)

    denq = den.reshape(H, DSTRIDE)[:, :N].T
    return _tc_norm(acc[0], acc[1], denq, bias.reshape(1, D))


# speedup vs baseline: 76.4939x; 1.0657x over previous
import functools

import jax
import jax.numpy as jnp
from jax import lax
from jax.experimental import pallas as pl
from jax.experimental.pallas import tpu as pltpu
from jax.experimental.pallas import tpu_sc as plsc

N = 10000
E = 320000
F = 128
H = 8
C = 32
D = H * C
HALF = D // 2
HPC = H // 2
NSUB = 16
K = 80
NCH = (E // NSUB) // K
PW = HPC * K
SA = 5
NSUPA = NCH // SA
SB = 10
NSUP = NCH // SB
BLK = 1000
RSL = 2512
DSTRIDE = 10048
DTOT = HPC * DSTRIDE


def _tc_project_body(x_ref, w_ref, as_ref, ad_ref, h0_ref, h1_ref,
                     als_ref, ald_ref):
    h = jnp.dot(x_ref[...], w_ref[...], preferred_element_type=jnp.float32)
    h0_ref[...] = h[:, :HALF]
    h1_ref[...] = h[:, HALF:]
    als_ref[...] = jnp.dot(h, as_ref[...], preferred_element_type=jnp.float32)
    ald_ref[...] = jnp.dot(h, ad_ref[...], preferred_element_type=jnp.float32)


def _tc_project(x, W, A_src, A_dst):
    return pl.pallas_call(
        _tc_project_body,
        grid=(N // BLK,),
        in_specs=[
            pl.BlockSpec((BLK, F), lambda i: (i, 0)),
            pl.BlockSpec((F, D), lambda i: (0, 0)),
            pl.BlockSpec((D, H), lambda i: (0, 0)),
            pl.BlockSpec((D, H), lambda i: (0, 0)),
        ],
        out_specs=[
            pl.BlockSpec((BLK, HALF), lambda i: (i, 0)),
            pl.BlockSpec((BLK, HALF), lambda i: (i, 0)),
            pl.BlockSpec((BLK, H), lambda i: (i, 0)),
            pl.BlockSpec((BLK, H), lambda i: (i, 0)),
        ],
        out_shape=[
            jax.ShapeDtypeStruct((N, HALF), jnp.float32),
            jax.ShapeDtypeStruct((N, HALF), jnp.float32),
            jax.ShapeDtypeStruct((N, H), jnp.float32),
            jax.ShapeDtypeStruct((N, H), jnp.float32),
        ],
    )(x, W, A_src, A_dst)


def _sc_mesh():
    return plsc.VectorSubcoreMesh(core_axis_name="c", subcore_axis_name="s")


@functools.cache
def _get_sc_weights():
    @functools.partial(
        pl.kernel,
        out_type=[
            jax.ShapeDtypeStruct((2 * E * HPC,), jnp.float32),
            jax.ShapeDtypeStruct((2 * DTOT,), jnp.float32),
            jax.ShapeDtypeStruct((2 * NSUB * DTOT,), jnp.float32),
        ],
        mesh=_sc_mesh(),
        scratch_types=[
            pltpu.VMEM((HPC * N,), jnp.float32),
            pltpu.VMEM((HPC * N,), jnp.float32),
            pltpu.VMEM((SA * K,), jnp.int32),
            pltpu.VMEM((SA * K,), jnp.int32),
            pltpu.VMEM((SA * K,), jnp.int32),
            pltpu.VMEM((SA * K,), jnp.int32),
            pltpu.VMEM((SA * PW,), jnp.float32),
            pltpu.VMEM((SA * PW,), jnp.float32),
            pltpu.VMEM((DTOT,), jnp.float32),
            pltpu.SemaphoreType.DMA,
            pltpu.SemaphoreType.DMA,
        ],
        compiler_params=pltpu.CompilerParams(needs_layout_passes=False),
    )
    def _sc_weights(alsT, aldT, srcF, dstF, zflat,
                    p_out, den_out, den_part,
                    as_v, ad_v, si0, di0, si1, di1, pc0, pc1, den_v,
                    sem_i, sem_p):
        c = lax.axis_index("c")
        s = lax.axis_index("s")

        pltpu.sync_copy(alsT.at[pl.ds(c * (HPC * N), HPC * N)], as_v)
        pltpu.sync_copy(aldT.at[pl.ds(c * (HPC * N), HPC * N)], ad_v)
        pltpu.sync_copy(zflat, den_v)

        sis = (si0, si1)
        dis = (di0, di1)
        pcs = (pc0, pc1)
        tbase = s * (NCH * K)
        pbase = (c * NSUB + s) * NCH * PW

        def issue_idx(u, b):
            tb = tbase + u * (SA * K)
            pltpu.async_copy(srcF.at[pl.ds(tb, SA * K)], sis[b], sem_i)
            pltpu.async_copy(dstF.at[pl.ds(tb, SA * K)], dis[b], sem_i)

        def chunk_maker(si_v, di_v, pch):
            def chunk(i2, carry):
                for g in range(K // 16):
                    s16 = si_v[pl.ds(i2 * K + g * 16, 16)]
                    d16 = di_v[pl.ds(i2 * K + g * 16, 16)]
                    for hh in range(HPC):
                        off = jnp.full((16,), hh * N, jnp.int32)
                        a_s = plsc.load_gather(as_v, [off + s16])
                        a_d = plsc.load_gather(ad_v, [off + d16])
                        z = a_s + a_d
                        e = jnp.maximum(z, 0.2 * z)
                        p = jnp.exp(e)
                        doff = jnp.full((16,), hh * DSTRIDE, jnp.int32)
                        plsc.addupdate_scatter(den_v, [doff + d16], p)
                        pch[pl.ds(i2 * PW + hh * K + g * 16, 16)] = p
                return carry
            return chunk

        def do_super(u, b, other):
            pltpu.make_async_copy(
                srcF.at[pl.ds(0, SA * K)], sis[b], sem_i).wait()
            pltpu.make_async_copy(
                dstF.at[pl.ds(0, SA * K)], dis[b], sem_i).wait()

            @pl.when(u < NSUPA - 1)
            def _():
                issue_idx(u + 1, other)

            @pl.when(u >= 2)
            def _():
                pltpu.make_async_copy(
                    pcs[b], p_out.at[pl.ds(0, SA * PW)], sem_p).wait()

            lax.fori_loop(0, SA, chunk_maker(sis[b], dis[b], pcs[b]), 0)
            pltpu.async_copy(
                pcs[b], p_out.at[pl.ds(pbase + u * (SA * PW), SA * PW)], sem_p)

        def super_pair(u2, carry):
            do_super(2 * u2, 0, 1)
            do_super(2 * u2 + 1, 1, 0)
            return carry

        issue_idx(0, 0)
        lax.fori_loop(0, NSUPA // 2, super_pair, 0)
        pltpu.make_async_copy(pc0, p_out.at[pl.ds(0, SA * PW)], sem_p).wait()
        pltpu.make_async_copy(pc1, p_out.at[pl.ds(0, SA * PW)], sem_p).wait()

        pltpu.sync_copy(den_v, den_part.at[pl.ds((c * NSUB + s) * DTOT, DTOT)])
        plsc.subcore_barrier()

        base = s * RSL
        pltpu.sync_copy(den_part.at[pl.ds(c * NSUB * DTOT + base, RSL)],
                        den_v.at[pl.ds(RSL, RSL)])

        def red(k, carry):
            pltpu.sync_copy(
                den_part.at[pl.ds((c * NSUB + k) * DTOT + base, RSL)],
                den_v.at[pl.ds(0, RSL)])
            for v in range(RSL // 16):
                sa = pl.ds(v * 16, 16)
                sb = pl.ds(RSL + v * 16, 16)
                den_v[sb] = den_v[sb] + den_v[sa]
            return carry

        lax.fori_loop(1, NSUB, red, 0)

        pltpu.sync_copy(den_v.at[pl.ds(RSL, RSL)],
                        den_out.at[pl.ds(c * DTOT + base, RSL)])

    return _sc_weights


@functools.cache
def _get_sc_scatter():
    @functools.partial(
        pl.kernel,
        out_type=jax.ShapeDtypeStruct((2, N, HALF), jnp.float32),
        mesh=_sc_mesh(),
        scratch_types=[
            pltpu.VMEM((SB * K,), jnp.int32),
            pltpu.VMEM((SB * K,), jnp.int32),
            pltpu.VMEM((SB, K), jnp.int32),
            pltpu.VMEM((SB, K), jnp.int32),
            pltpu.VMEM((K, HALF), jnp.float32),
            pltpu.VMEM((K, HALF), jnp.float32),
            pltpu.VMEM((K, HALF), jnp.float32),
            pltpu.VMEM((K, HALF), jnp.float32),
            pltpu.VMEM((SB * PW,), jnp.float32),
            pltpu.VMEM_SHARED((N, HALF), jnp.float32),
            pltpu.SemaphoreType.DMA,
            pltpu.SemaphoreType.DMA,
            pltpu.SemaphoreType.DMA,
        ],
        compiler_params=pltpu.CompilerParams(needs_layout_passes=False),
    )
    def _sc_scatter(h0, h1, srcF, dstSB, p_in, zrows,
                    acc_out,
                    si0, si1, di0, di1, gb0, gb1, sb0, sb1, pch, acc,
                    sem_g, sem_s, sem_i):
        c = lax.axis_index("c")
        s = lax.axis_index("s")

        @pl.when(s == 0)
        def _():
            pltpu.sync_copy(zrows, acc)

        plsc.subcore_barrier()

        sis = (si0, si1)
        dis = (di0, di1)
        tbase = s * (NCH * K)
        pbase = (c * NSUB + s) * NCH * PW

        def issue_idx(u, b):
            pltpu.async_copy(
                srcF.at[pl.ds(tbase + u * (SB * K), SB * K)], sis[b], sem_i)
            pltpu.async_copy(dstSB.at[s * NSUP + u], dis[b], sem_i)

        def process(h_ref):
            gbufs = (gb0, gb1)
            sbufs = (sb0, sb1)

            def do_chunk(t, b, other, si_v, di_v):
                pltpu.make_async_copy(
                    h_ref.at[si_v.at[pl.ds(0, K)]], gbufs[b], sem_g).wait()

                @pl.when(t < SB - 1)
                def _():
                    pltpu.async_copy(
                        h_ref.at[si_v.at[pl.ds((t + 1) * K, K)]],
                        gbufs[other], sem_g)

                @pl.when(t >= 2)
                def _():
                    pltpu.make_async_copy(
                        sbufs[b], acc.at[di_v.at[0]], sem_s).wait()

                def scale_group(g, carry3):
                    p_list = [pch[pl.ds(t * PW + hh * K + g * 16, 16)]
                              for hh in range(HPC)]
                    for j in range(16):
                        ej = g * 16 + j
                        for hh in range(HPC):
                            pj = p_list[hh][j]
                            sl0 = pl.ds(hh * 32, 16)
                            sl1 = pl.ds(hh * 32 + 16, 16)
                            sbufs[b][ej, sl0] = gbufs[b][ej, sl0] * pj
                            sbufs[b][ej, sl1] = gbufs[b][ej, sl1] * pj
                    return carry3

                lax.fori_loop(0, K // 16, scale_group, 0)

                pltpu.async_copy(sbufs[b], acc.at[di_v.at[t]], sem_s, add=True)

            def do_super(u, ib, other):
                pltpu.async_copy(
                    p_in.at[pl.ds(pbase + u * (SB * PW), SB * PW)], pch, sem_i)

                @pl.when(u >= 1)
                def _():
                    pltpu.make_async_copy(
                        sb0, acc.at[dis[ib].at[0]], sem_s).wait()
                    pltpu.make_async_copy(
                        sb1, acc.at[dis[ib].at[0]], sem_s).wait()

                pltpu.make_async_copy(
                    srcF.at[pl.ds(0, SB * K)], sis[ib], sem_i).wait()
                pltpu.make_async_copy(
                    dstSB.at[0], dis[ib], sem_i).wait()

                @pl.when(u < NSUP - 1)
                def _():
                    issue_idx(u + 1, other)

                pltpu.make_async_copy(
                    p_in.at[pl.ds(0, SB * PW)], pch, sem_i).wait()

                pltpu.async_copy(
                    h_ref.at[sis[ib].at[pl.ds(0, K)]], gb0, sem_g)

                def chunk_pair(t2, carry2):
                    do_chunk(2 * t2, 0, 1, sis[ib], dis[ib])
                    do_chunk(2 * t2 + 1, 1, 0, sis[ib], dis[ib])
                    return carry2

                lax.fori_loop(0, SB // 2, chunk_pair, 0)

            def super_pair(u2, carry):
                do_super(2 * u2, 0, 1)
                do_super(2 * u2 + 1, 1, 0)
                return carry

            issue_idx(0, 0)
            lax.fori_loop(0, NSUP // 2, super_pair, 0)
            do_super(jnp.int32(NSUP - 1), 0, 1)
            pltpu.make_async_copy(sb0, acc.at[dis[0].at[0]], sem_s).wait()
            pltpu.make_async_copy(sb1, acc.at[dis[0].at[0]], sem_s).wait()

        @pl.when(c == 0)
        def _():
            process(h0)

        @pl.when(c == 1)
        def _():
            process(h1)

        plsc.subcore_barrier()

        @pl.when(s == 0)
        def _():
            pltpu.sync_copy(acc, acc_out.at[c])

    return _sc_scatter


def _tc_norm_body(a0_ref, a1_ref, dq_ref, b_ref, o_ref):
    a0 = a0_ref[...]
    a1 = a1_ref[...]
    dq = dq_ref[...]
    parts = []
    for half, a in enumerate((a0, a1)):
        for hh in range(HPC):
            g = half * HPC + hh
            num = a[:, hh * C:(hh + 1) * C]
            den = dq[:, g:g + 1]
            parts.append(num / (den + 1e-16))
    o_ref[...] = jnp.concatenate(parts, axis=1) + b_ref[...]


def _tc_norm(acc0, acc1, denq, bias2d):
    return pl.pallas_call(
        _tc_norm_body,
        grid=(N // BLK,),
        in_specs=[
            pl.BlockSpec((BLK, HALF), lambda i: (i, 0)),
            pl.BlockSpec((BLK, HALF), lambda i: (i, 0)),
            pl.BlockSpec((BLK, H), lambda i: (i, 0)),
            pl.BlockSpec((1, D), lambda i: (0, 0)),
        ],
        out_specs=pl.BlockSpec((BLK, D), lambda i: (i, 0)),
        out_shape=jax.ShapeDtypeStruct((N, D), jnp.float32),
    )(acc0, acc1, denq, bias2d)


def kernel(x, edge_index, W, att_src, att_dst, bias):
    eye = jnp.eye(H, dtype=jnp.float32)
    A_src = (att_src[:, :, None] * eye[:, None, :]).reshape(D, H)
    A_dst = (att_dst[:, :, None] * eye[:, None, :]).reshape(D, H)

    h0, h1, als, ald = _tc_project(x, W, A_src, A_dst)
    alsT = als.T.reshape(-1)
    aldT = ald.T.reshape(-1)

    srcF = edge_index[0]
    dstF = edge_index[1]
    dstSB = dstF.reshape(NSUB * NSUP, SB, K)
    zrows = jnp.zeros((N, HALF), jnp.float32)
    zflat = jnp.zeros((DTOT,), jnp.float32)

    p_all, den, _ = _get_sc_weights()(alsT, aldT, srcF, dstF, zflat)
    acc = _get_sc_scatter()(h0, h1, srcF, dstSB, p_all, zrows)

    denq = den.reshape(H, DSTRIDE)[:, :N].T
    return _tc_norm(acc[0], acc[1], denq, bias.reshape(1, D))


# speedup vs baseline: 76.9276x; 1.0057x over previous
import functools

import jax
import jax.numpy as jnp
from jax import lax
from jax.experimental import pallas as pl
from jax.experimental.pallas import tpu as pltpu
from jax.experimental.pallas import tpu_sc as plsc

N = 10000
E = 320000
F = 128
H = 8
C = 32
D = H * C
HALF = D // 2
HPC = H // 2
NSUB = 16
K = 80
NCH = (E // NSUB) // K
PW = HPC * K
SA = 5
NSUPA = NCH // SA
SB = 10
NSUP = NCH // SB
BLK = 1000
RSL = 2512
DSTRIDE = 10048
DTOT = HPC * DSTRIDE


def _tc_project_body(x_ref, w_ref, as_ref, ad_ref, h0_ref, h1_ref,
                     als_ref, ald_ref):
    h = jnp.dot(x_ref[...], w_ref[...], preferred_element_type=jnp.float32)
    h0_ref[...] = h[:, :HALF]
    h1_ref[...] = h[:, HALF:]
    als_ref[...] = jnp.dot(h, as_ref[...], preferred_element_type=jnp.float32)
    ald_ref[...] = jnp.dot(h, ad_ref[...], preferred_element_type=jnp.float32)


def _tc_project(x, W, A_src, A_dst):
    return pl.pallas_call(
        _tc_project_body,
        grid=(N // BLK,),
        in_specs=[
            pl.BlockSpec((BLK, F), lambda i: (i, 0)),
            pl.BlockSpec((F, D), lambda i: (0, 0)),
            pl.BlockSpec((D, H), lambda i: (0, 0)),
            pl.BlockSpec((D, H), lambda i: (0, 0)),
        ],
        out_specs=[
            pl.BlockSpec((BLK, HALF), lambda i: (i, 0)),
            pl.BlockSpec((BLK, HALF), lambda i: (i, 0)),
            pl.BlockSpec((BLK, H), lambda i: (i, 0)),
            pl.BlockSpec((BLK, H), lambda i: (i, 0)),
        ],
        out_shape=[
            jax.ShapeDtypeStruct((N, HALF), jnp.float32),
            jax.ShapeDtypeStruct((N, HALF), jnp.float32),
            jax.ShapeDtypeStruct((N, H), jnp.float32),
            jax.ShapeDtypeStruct((N, H), jnp.float32),
        ],
    )(x, W, A_src, A_dst)


def _sc_mesh():
    return plsc.VectorSubcoreMesh(core_axis_name="c", subcore_axis_name="s")


@functools.cache
def _get_sc_weights():
    @functools.partial(
        pl.kernel,
        out_type=[
            jax.ShapeDtypeStruct((2 * E * HPC,), jnp.float32),
            jax.ShapeDtypeStruct((2 * DTOT,), jnp.float32),
            jax.ShapeDtypeStruct((2 * NSUB * DTOT,), jnp.float32),
        ],
        mesh=_sc_mesh(),
        scratch_types=[
            pltpu.VMEM((HPC * N,), jnp.float32),
            pltpu.VMEM((HPC * N,), jnp.float32),
            pltpu.VMEM((SA * K,), jnp.int32),
            pltpu.VMEM((SA * K,), jnp.int32),
            pltpu.VMEM((SA * K,), jnp.int32),
            pltpu.VMEM((SA * K,), jnp.int32),
            pltpu.VMEM((SA * PW,), jnp.float32),
            pltpu.VMEM((SA * PW,), jnp.float32),
            pltpu.VMEM((DTOT,), jnp.float32),
            pltpu.SemaphoreType.DMA,
            pltpu.SemaphoreType.DMA,
        ],
        compiler_params=pltpu.CompilerParams(needs_layout_passes=False),
    )
    def _sc_weights(alsT, aldT, srcF, dstF, zflat,
                    p_out, den_out, den_part,
                    as_v, ad_v, si0, di0, si1, di1, pc0, pc1, den_v,
                    sem_i, sem_p):
        c = lax.axis_index("c")
        s = lax.axis_index("s")

        pltpu.sync_copy(alsT.at[pl.ds(c * (HPC * N), HPC * N)], as_v)
        pltpu.sync_copy(aldT.at[pl.ds(c * (HPC * N), HPC * N)], ad_v)
        pltpu.sync_copy(zflat, den_v)

        sis = (si0, si1)
        dis = (di0, di1)
        pcs = (pc0, pc1)
        tbase = s * (NCH * K)
        pbase = (c * NSUB + s) * NCH * PW

        def issue_idx(u, b):
            tb = tbase + u * (SA * K)
            pltpu.async_copy(srcF.at[pl.ds(tb, SA * K)], sis[b], sem_i)
            pltpu.async_copy(dstF.at[pl.ds(tb, SA * K)], dis[b], sem_i)

        def chunk_maker(si_v, di_v, pch):
            def chunk(i2, carry):
                for g in range(K // 16):
                    s16 = si_v[pl.ds(i2 * K + g * 16, 16)]
                    d16 = di_v[pl.ds(i2 * K + g * 16, 16)]
                    for hh in range(HPC):
                        off = jnp.full((16,), hh * N, jnp.int32)
                        a_s = plsc.load_gather(as_v, [off + s16])
                        a_d = plsc.load_gather(ad_v, [off + d16])
                        z = a_s + a_d
                        e = jnp.maximum(z, 0.2 * z)
                        p = jnp.exp(e)
                        doff = jnp.full((16,), hh * DSTRIDE, jnp.int32)
                        plsc.addupdate_scatter(den_v, [doff + d16], p)
                        pch[pl.ds(i2 * PW + hh * K + g * 16, 16)] = p
                return carry
            return chunk

        def do_super(u, b, other):
            pltpu.make_async_copy(
                srcF.at[pl.ds(0, SA * K)], sis[b], sem_i).wait()
            pltpu.make_async_copy(
                dstF.at[pl.ds(0, SA * K)], dis[b], sem_i).wait()

            @pl.when(u < NSUPA - 1)
            def _():
                issue_idx(u + 1, other)

            @pl.when(u >= 2)
            def _():
                pltpu.make_async_copy(
                    pcs[b], p_out.at[pl.ds(0, SA * PW)], sem_p).wait()

            lax.fori_loop(0, SA, chunk_maker(sis[b], dis[b], pcs[b]), 0)
            pltpu.async_copy(
                pcs[b], p_out.at[pl.ds(pbase + u * (SA * PW), SA * PW)], sem_p)

        def super_pair(u2, carry):
            do_super(2 * u2, 0, 1)
            do_super(2 * u2 + 1, 1, 0)
            return carry

        issue_idx(0, 0)
        lax.fori_loop(0, NSUPA // 2, super_pair, 0)
        pltpu.make_async_copy(pc0, p_out.at[pl.ds(0, SA * PW)], sem_p).wait()
        pltpu.make_async_copy(pc1, p_out.at[pl.ds(0, SA * PW)], sem_p).wait()

        pltpu.sync_copy(den_v, den_part.at[pl.ds((c * NSUB + s) * DTOT, DTOT)])
        plsc.subcore_barrier()

        base = s * RSL
        pltpu.sync_copy(den_part.at[pl.ds(c * NSUB * DTOT + base, RSL)],
                        den_v.at[pl.ds(RSL, RSL)])
        pltpu.async_copy(
            den_part.at[pl.ds((c * NSUB + 1) * DTOT + base, RSL)],
            den_v.at[pl.ds(0, RSL)], sem_i)

        def red(k, roff, other_off):
            pltpu.make_async_copy(
                den_part.at[pl.ds(base, RSL)],
                den_v.at[pl.ds(roff, RSL)], sem_i).wait()

            @pl.when(k < NSUB - 1)
            def _():
                pltpu.async_copy(
                    den_part.at[pl.ds((c * NSUB + k + 1) * DTOT + base, RSL)],
                    den_v.at[pl.ds(other_off, RSL)], sem_i)

            def add_vec(v, carry4):
                sa = pl.ds(roff + v * 16, 16)
                sb = pl.ds(RSL + v * 16, 16)
                den_v[sb] = den_v[sb] + den_v[sa]
                return carry4

            lax.fori_loop(0, RSL // 16, add_vec, 0)

        def red_pair(t2, carry):
            red(2 * t2 + 1, 0, 2 * RSL)
            red(2 * t2 + 2, 2 * RSL, 0)
            return carry

        lax.fori_loop(0, (NSUB - 2) // 2, red_pair, 0)
        red(jnp.int32(NSUB - 1), 0, 2 * RSL)

        pltpu.sync_copy(den_v.at[pl.ds(RSL, RSL)],
                        den_out.at[pl.ds(c * DTOT + base, RSL)])

    return _sc_weights


@functools.cache
def _get_sc_scatter():
    @functools.partial(
        pl.kernel,
        out_type=jax.ShapeDtypeStruct((2, N, HALF), jnp.float32),
        mesh=_sc_mesh(),
        scratch_types=[
            pltpu.VMEM((SB * K,), jnp.int32),
            pltpu.VMEM((SB * K,), jnp.int32),
            pltpu.VMEM((SB, K), jnp.int32),
            pltpu.VMEM((SB, K), jnp.int32),
            pltpu.VMEM((K, HALF), jnp.float32),
            pltpu.VMEM((K, HALF), jnp.float32),
            pltpu.VMEM((K, HALF), jnp.float32),
            pltpu.VMEM((K, HALF), jnp.float32),
            pltpu.VMEM((SB * PW,), jnp.float32),
            pltpu.VMEM_SHARED((N, HALF), jnp.float32),
            pltpu.SemaphoreType.DMA,
            pltpu.SemaphoreType.DMA,
            pltpu.SemaphoreType.DMA,
        ],
        compiler_params=pltpu.CompilerParams(needs_layout_passes=False),
    )
    def _sc_scatter(h0, h1, srcF, dstSB, p_in, zrows,
                    acc_out,
                    si0, si1, di0, di1, gb0, gb1, sb0, sb1, pch, acc,
                    sem_g, sem_s, sem_i):
        c = lax.axis_index("c")
        s = lax.axis_index("s")

        @pl.when(s == 0)
        def _():
            pltpu.sync_copy(zrows, acc)

        plsc.subcore_barrier()

        sis = (si0, si1)
        dis = (di0, di1)
        tbase = s * (NCH * K)
        pbase = (c * NSUB + s) * NCH * PW

        def issue_idx(u, b):
            pltpu.async_copy(
                srcF.at[pl.ds(tbase + u * (SB * K), SB * K)], sis[b], sem_i)
            pltpu.async_copy(dstSB.at[s * NSUP + u], dis[b], sem_i)

        def process(h_ref):
            gbufs = (gb0, gb1)
            sbufs = (sb0, sb1)

            def do_chunk(t, b, other, si_v, di_v):
                pltpu.make_async_copy(
                    h_ref.at[si_v.at[pl.ds(0, K)]], gbufs[b], sem_g).wait()

                @pl.when(t < SB - 1)
                def _():
                    pltpu.async_copy(
                        h_ref.at[si_v.at[pl.ds((t + 1) * K, K)]],
                        gbufs[other], sem_g)

                @pl.when(t >= 2)
                def _():
                    pltpu.make_async_copy(
                        sbufs[b], acc.at[di_v.at[0]], sem_s).wait()

                def scale_group(g, carry3):
                    p_list = [pch[pl.ds(t * PW + hh * K + g * 16, 16)]
                              for hh in range(HPC)]
                    for j in range(16):
                        ej = g * 16 + j
                        for hh in range(HPC):
                            pj = p_list[hh][j]
                            sl0 = pl.ds(hh * 32, 16)
                            sl1 = pl.ds(hh * 32 + 16, 16)
                            sbufs[b][ej, sl0] = gbufs[b][ej, sl0] * pj
                            sbufs[b][ej, sl1] = gbufs[b][ej, sl1] * pj
                    return carry3

                lax.fori_loop(0, K // 16, scale_group, 0)

                pltpu.async_copy(sbufs[b], acc.at[di_v.at[t]], sem_s, add=True)

            def do_super(u, ib, other):
                pltpu.async_copy(
                    p_in.at[pl.ds(pbase + u * (SB * PW), SB * PW)], pch, sem_i)

                @pl.when(u >= 1)
                def _():
                    pltpu.make_async_copy(
                        sb0, acc.at[dis[ib].at[0]], sem_s).wait()
                    pltpu.make_async_copy(
                        sb1, acc.at[dis[ib].at[0]], sem_s).wait()

                pltpu.make_async_copy(
                    srcF.at[pl.ds(0, SB * K)], sis[ib], sem_i).wait()
                pltpu.make_async_copy(
                    dstSB.at[0], dis[ib], sem_i).wait()

                @pl.when(u < NSUP - 1)
                def _():
                    issue_idx(u + 1, other)

                pltpu.make_async_copy(
                    p_in.at[pl.ds(0, SB * PW)], pch, sem_i).wait()

                pltpu.async_copy(
                    h_ref.at[sis[ib].at[pl.ds(0, K)]], gb0, sem_g)

                def chunk_pair(t2, carry2):
                    do_chunk(2 * t2, 0, 1, sis[ib], dis[ib])
                    do_chunk(2 * t2 + 1, 1, 0, sis[ib], dis[ib])
                    return carry2

                lax.fori_loop(0, SB // 2, chunk_pair, 0)

            def super_pair(u2, carry):
                do_super(2 * u2, 0, 1)
                do_super(2 * u2 + 1, 1, 0)
                return carry

            issue_idx(0, 0)
            lax.fori_loop(0, NSUP // 2, super_pair, 0)
            do_super(jnp.int32(NSUP - 1), 0, 1)
            pltpu.make_async_copy(sb0, acc.at[dis[0].at[0]], sem_s).wait()
            pltpu.make_async_copy(sb1, acc.at[dis[0].at[0]], sem_s).wait()

        @pl.when(c == 0)
        def _():
            process(h0)

        @pl.when(c == 1)
        def _():
            process(h1)

        plsc.subcore_barrier()

        @pl.when(s == 0)
        def _():
            pltpu.sync_copy(acc, acc_out.at[c])

    return _sc_scatter


def _tc_norm_body(a0_ref, a1_ref, dq_ref, b_ref, o_ref):
    a0 = a0_ref[...]
    a1 = a1_ref[...]
    dq = dq_ref[...]
    parts = []
    for half, a in enumerate((a0, a1)):
        for hh in range(HPC):
            g = half * HPC + hh
            num = a[:, hh * C:(hh + 1) * C]
            den = dq[:, g:g + 1]
            parts.append(num / (den + 1e-16))
    o_ref[...] = jnp.concatenate(parts, axis=1) + b_ref[...]


def _tc_norm(acc0, acc1, denq, bias2d):
    return pl.pallas_call(
        _tc_norm_body,
        grid=(N // BLK,),
        in_specs=[
            pl.BlockSpec((BLK, HALF), lambda i: (i, 0)),
            pl.BlockSpec((BLK, HALF), lambda i: (i, 0)),
            pl.BlockSpec((BLK, H), lambda i: (i, 0)),
            pl.BlockSpec((1, D), lambda i: (0, 0)),
        ],
        out_specs=pl.BlockSpec((BLK, D), lambda i: (i, 0)),
        out_shape=jax.ShapeDtypeStruct((N, D), jnp.float32),
    )(acc0, acc1, denq, bias2d)


def kernel(x, edge_index, W, att_src, att_dst, bias):
    eye = jnp.eye(H, dtype=jnp.float32)
    A_src = (att_src[:, :, None] * eye[:, None, :]).reshape(D, H)
    A_dst = (att_dst[:, :, None] * eye[:, None, :]).reshape(D, H)

    h0, h1, als, ald = _tc_project(x, W, A_src, A_dst)
    alsT = als.T.reshape(-1)
    aldT = ald.T.reshape(-1)

    srcF = edge_index[0]
    dstF = edge_index[1]
    dstSB = dstF.reshape(NSUB * NSUP, SB, K)
    zrows = jnp.zeros((N, HALF), jnp.float32)
    zflat = jnp.zeros((DTOT,), jnp.float32)

    p_all, den, _ = _get_sc_weights()(alsT, aldT, srcF, dstF, zflat)
    acc = _get_sc_scatter()(h0, h1, srcF, dstSB, p_all, zrows)

    denq = den.reshape(H, DSTRIDE)[:, :N].T
    return _tc_norm(acc[0], acc[1], denq, bias.reshape(1, D))
